# baseline (speedup 1.0000x reference)
import functools

import jax
import jax.numpy as jnp
from jax.experimental import pallas as pl


def _nt(x, w):
    return jax.lax.dot_general(x, w, (((1,), (1,)), ((), ())),
                               preferred_element_type=jnp.float32)


def _transform_kernel(s_ref, Win_ref, bin_ref, Wout_ref, bout_ref,
                      sin_ref, sout_ref):
    s = s_ref[...]
    sin_ref[...] = _nt(s, Win_ref[...]) + bin_ref[...]
    sout_ref[...] = _nt(s, Wout_ref[...]) + bout_ref[...]


def _gate_kernel(A_ref, sin_ref, sout_ref, s_ref, Wr_ref, br_ref, Wz_ref,
                 bz_ref, Wt_ref, bt_ref, out_ref, *, d):
    a_in = jnp.dot(A_ref[0], sin_ref[...], preferred_element_type=jnp.float32)
    a_out = jnp.dot(A_ref[1], sout_ref[...], preferred_element_type=jnp.float32)
    s_blk = s_ref[...]

    def split_nt(x1, x2, x3, W, b):
        return ((_nt(x1, W[:, :d]) + _nt(x2, W[:, d:2 * d]))
                + _nt(x3, W[:, 2 * d:])) + b

    r = jax.nn.sigmoid(split_nt(a_in, a_out, s_blk, Wr_ref[...], br_ref[...]))
    z = jax.nn.sigmoid(split_nt(a_in, a_out, s_blk, Wz_ref[...], bz_ref[...]))
    h = jnp.tanh(split_nt(a_in, a_out, r * s_blk, Wt_ref[...], bt_ref[...]))
    out_ref[...] = (1.0 - z) * s_blk + z * h


N_PROP_STEPS = 5
_BM = 256


def kernel(prop_state, A, W_in, b_in, W_out, b_out, W_r, b_r, W_z, b_z, W_t, b_t):
    n, d = prop_state.shape
    bm = _BM

    b2 = lambda v: v.reshape(1, d)
    bin2, bout2, br2, bz2, bt2 = b2(b_in), b2(b_out), b2(b_r), b2(b_z), b2(b_t)

    transform = pl.pallas_call(
        _transform_kernel,
        out_shape=[
            jax.ShapeDtypeStruct((n, d), jnp.float32),
            jax.ShapeDtypeStruct((n, d), jnp.float32),
        ],
    )

    gate = pl.pallas_call(
        functools.partial(_gate_kernel, d=d),
        grid=(n // bm,),
        in_specs=[
            pl.BlockSpec((2, bm, n), lambda i: (0, i, 0)),
            pl.BlockSpec((n, d), lambda i: (0, 0)),
            pl.BlockSpec((n, d), lambda i: (0, 0)),
            pl.BlockSpec((bm, d), lambda i: (i, 0)),
            pl.BlockSpec((d, 3 * d), lambda i: (0, 0)),
            pl.BlockSpec((1, d), lambda i: (0, 0)),
            pl.BlockSpec((d, 3 * d), lambda i: (0, 0)),
            pl.BlockSpec((1, d), lambda i: (0, 0)),
            pl.BlockSpec((d, 3 * d), lambda i: (0, 0)),
            pl.BlockSpec((1, d), lambda i: (0, 0)),
        ],
        out_specs=pl.BlockSpec((bm, d), lambda i: (i, 0)),
        out_shape=jax.ShapeDtypeStruct((n, d), jnp.float32),
    )

    s = prop_state
    for _ in range(N_PROP_STEPS):
        state_in, state_out = transform(s, W_in, bin2, W_out, bout2)
        s = gate(A, state_in, state_out, s, W_r, br2, W_z, bz2, W_t, bt2)
    return s


# speedup vs baseline: 1.1628x; 1.1628x over previous
import functools

import jax
import jax.numpy as jnp
from jax.experimental import pallas as pl
from jax.experimental.pallas import tpu as pltpu


def _nt(x, w):
    return jax.lax.dot_general(x, w, (((1,), (1,)), ((), ())),
                               preferred_element_type=jnp.float32)


def _step_kernel(s_ref, A_ref, Win_ref, bin_ref, Wout_ref, bout_ref,
                 Wr_ref, br_ref, Wz_ref, bz_ref, Wt_ref, bt_ref,
                 out_ref, sin_scr, sout_scr, *, bm, d):
    i = pl.program_id(0)

    @pl.when(i == 0)
    def _transform():
        s = s_ref[...]
        sin_scr[...] = _nt(s, Win_ref[...]) + bin_ref[...]
        sout_scr[...] = _nt(s, Wout_ref[...]) + bout_ref[...]

    a_in = jnp.dot(A_ref[0], sin_scr[...], preferred_element_type=jnp.float32)
    a_out = jnp.dot(A_ref[1], sout_scr[...], preferred_element_type=jnp.float32)
    s_blk = s_ref[pl.ds(i * bm, bm), :]
    a = jnp.concatenate([a_in, a_out, s_blk], axis=1)
    r = jax.nn.sigmoid(_nt(a, Wr_ref[...]) + br_ref[...])
    z = jax.nn.sigmoid(_nt(a, Wz_ref[...]) + bz_ref[...])
    ji = jnp.concatenate([a_in, a_out, r * s_blk], axis=1)
    h = jnp.tanh(_nt(ji, Wt_ref[...]) + bt_ref[...])
    out_ref[...] = (1.0 - z) * s_blk + z * h


N_PROP_STEPS = 5
_BM = 256


def kernel(prop_state, A, W_in, b_in, W_out, b_out, W_r, b_r, W_z, b_z, W_t, b_t):
    n, d = prop_state.shape
    bm = _BM

    b2 = lambda v: v.reshape(1, d)
    bin2, bout2, br2, bz2, bt2 = b2(b_in), b2(b_out), b2(b_r), b2(b_z), b2(b_t)

    full = lambda shape: pl.BlockSpec(shape, lambda i: tuple(0 for _ in shape))
    step = pl.pallas_call(
        functools.partial(_step_kernel, bm=bm, d=d),
        grid=(n // bm,),
        in_specs=[
            full((n, d)),
            pl.BlockSpec((2, bm, n), lambda i: (0, i, 0)),
            full((d, d)), full((1, d)),
            full((d, d)), full((1, d)),
            full((d, 3 * d)), full((1, d)),
            full((d, 3 * d)), full((1, d)),
            full((d, 3 * d)), full((1, d)),
        ],
        out_specs=pl.BlockSpec((bm, d), lambda i: (i, 0)),
        out_shape=jax.ShapeDtypeStruct((n, d), jnp.float32),
        scratch_shapes=[pltpu.VMEM((n, d), jnp.float32),
                        pltpu.VMEM((n, d), jnp.float32)],
    )

    s = prop_state
    for _ in range(N_PROP_STEPS):
        s = step(s, A, W_in, bin2, W_out, bout2, W_r, br2, W_z, bz2, W_t, bt2)
    return s
